# baseline (speedup 1.0000x reference)
import jax
import jax.numpy as jnp
import numpy as np
from jax.experimental import pallas as pl
from jax.experimental.pallas import tpu as pltpu


def _decoder_body(T, V, Vp, DIN, H, ET,
                  w_ref, ins_ref, w1s_ref, w1r_ref, b1_ref, w2_ref, b2_ref,
                  hr_ref, hi_ref, hh_ref,
                  irw_ref, irb_ref, iiw_ref, iib_ref, inw_ref, inb_ref,
                  o1w_ref, o1b_ref, o2w_ref, o2b_ref, o3w_ref, o3b_ref,
                  out_ref):
    P = V * Vp
    H2 = 2 * H
    f32 = jnp.float32

    inv_norm = 1.0 / ((ET - 1.0) * (V - 1.0))
    bf16 = jnp.bfloat16
    dot = lambda a, b: jnp.dot(a, b, preferred_element_type=f32)

    wp = [jnp.concatenate(
              [jnp.broadcast_to(w_ref[bb, :, et:et + 1], (P, H))
               for bb in range(2)], axis=1)
          for et in range(ET)]

    ins = ins_ref[0]
    hidden = jnp.zeros((Vp, H2), dtype=f32)

    for t in range(T):
        m2w = jnp.zeros((P, H2), dtype=bf16)
        hid_bf = hidden.astype(bf16)
        for et in range(1, ET):
            s_part = dot(hid_bf, w1s_ref[et]).astype(bf16)
            a_part = dot(hid_bf, w1r_ref[et]).astype(bf16) + b1_ref[et]
            pre = s_part[:V][:, None, :] + a_part[None, :, :]
            m = jnp.tanh(pre).reshape(P, H2)
            m2 = jnp.tanh(dot(m, w2_ref[et]).astype(bf16) + b2_ref[et])
            m2w = m2w + m2 * wp[et]
        agg = jnp.sum(m2w.reshape(V, Vp, H2).astype(f32),
                      axis=0) * inv_norm

        inp_r = dot(ins, irw_ref[...]) + irb_ref[0]
        inp_i = dot(ins, iiw_ref[...]) + iib_ref[0]
        inp_n = dot(ins, inw_ref[...]) + inb_ref[0]
        r = jax.nn.sigmoid(inp_r + dot(agg, hr_ref[...]))
        ig = jax.nn.sigmoid(inp_i + dot(agg, hi_ref[...]))
        n = jnp.tanh(inp_n + r * dot(agg, hh_ref[...]))
        hidden = (1.0 - ig) * n + ig * hidden

        p = jax.nn.relu(dot(hidden, o1w_ref[...]) + o1b_ref[0])
        p = jax.nn.relu(dot(p, o2w_ref[...]) + o2b_ref[0])
        p = dot(p, o3w_ref[...]) + o3b_ref[0]
        pred = ins + p
        out_ref[0, t] = pred[:V, :DIN]
        out_ref[1, t] = pred[:V, DIN:]
        ins = pred


def kernel(inputs, sampled_edges, msg_fc1_w, msg_fc1_b, msg_fc2_w,
           msg_fc2_b, hidden_r_w, hidden_i_w, hidden_h_w, input_r_w,
           input_r_b, input_i_w, input_i_b, input_n_w, input_n_b,
           out_fc1_w, out_fc1_b, out_fc2_w, out_fc2_b, out_fc3_w,
           out_fc3_b):
    B, T, V, DIN = inputs.shape
    H = hidden_r_w.shape[0]
    ET = msg_fc1_w.shape[0]
    Vp = (V + 7) // 8 * 8
    P = V * Vp
    NB = 2

    adj = np.ones((V, V)) - np.eye(V)
    send_np, recv_np = np.where(adj)
    p_idx = jnp.asarray(send_np * Vp + recv_np, dtype=jnp.int32)
    w_dense = jnp.zeros((B, P, ET), dtype=jnp.float32)
    w_dense = w_dense.at[:, p_idx, :].set(sampled_edges)
    w_dense = w_dense.astype(jnp.bfloat16)

    ins0 = jnp.pad(inputs[:, 0], ((0, 0), (0, Vp - V), (0, 0)))
    ins0 = ins0.reshape(B // NB, NB, Vp, DIN).transpose(0, 2, 1, 3)
    ins0 = ins0.reshape(B // NB, Vp, NB * DIN)

    def bd(m):
        z = jnp.zeros_like(m)
        return jnp.concatenate(
            [jnp.concatenate([m, z], axis=1),
             jnp.concatenate([z, m], axis=1)], axis=0)

    bf16 = jnp.bfloat16
    w1s_bd = jnp.stack([bd(msg_fc1_w[et, H:, :]) for et in range(ET)]).astype(bf16)
    w1r_bd = jnp.stack([bd(msg_fc1_w[et, :H, :]) for et in range(ET)]).astype(bf16)
    b1_2 = jnp.tile(msg_fc1_b, (1, NB)).astype(bf16)
    w2_bd = jnp.stack([bd(msg_fc2_w[et]) for et in range(ET)]).astype(bf16)
    b2_2 = jnp.tile(msg_fc2_b, (1, NB)).astype(bf16)
    hr_bd, hi_bd, hh_bd = bd(hidden_r_w), bd(hidden_i_w), bd(hidden_h_w)
    irw_bd, iiw_bd, inw_bd = bd(input_r_w), bd(input_i_w), bd(input_n_w)
    o1w_bd, o2w_bd, o3w_bd = bd(out_fc1_w), bd(out_fc2_w), bd(out_fc3_w)
    t2 = lambda b: jnp.tile(b.reshape(1, -1), (1, NB))

    def body(*refs):
        _decoder_body(T, V, Vp, DIN, H, ET, *refs)

    rep3 = lambda shp: pl.BlockSpec(shp, lambda b: (0, 0, 0))
    rep2 = lambda shp: pl.BlockSpec(shp, lambda b: (0, 0))
    H2 = 2 * H
    D2 = 2 * DIN

    out = pl.pallas_call(
        body,
        grid=(B // NB,),
        in_specs=[
            pl.BlockSpec((NB, P, ET), lambda b: (b, 0, 0)),
            pl.BlockSpec((1, Vp, D2), lambda b: (b, 0, 0)),
            rep3((ET, H2, H2)),
            rep3((ET, H2, H2)),
            rep2((ET, H2)),
            rep3((ET, H2, H2)),
            rep2((ET, H2)),
            rep2((H2, H2)), rep2((H2, H2)), rep2((H2, H2)),
            rep2((D2, H2)), rep2((1, H2)),
            rep2((D2, H2)), rep2((1, H2)),
            rep2((D2, H2)), rep2((1, H2)),
            rep2((H2, H2)), rep2((1, H2)),
            rep2((H2, H2)), rep2((1, H2)),
            rep2((H2, D2)), rep2((1, D2)),
        ],
        out_specs=pl.BlockSpec((NB, T, V, DIN), lambda b: (b, 0, 0, 0)),
        out_shape=jax.ShapeDtypeStruct((B, T, V, DIN), jnp.float32),
        compiler_params=pltpu.CompilerParams(
            dimension_semantics=("parallel",)),
    )(w_dense, ins0, w1s_bd, w1r_bd, b1_2, w2_bd, b2_2,
      hr_bd, hi_bd, hh_bd,
      irw_bd, t2(input_r_b), iiw_bd, t2(input_i_b), inw_bd, t2(input_n_b),
      o1w_bd, t2(out_fc1_b), o2w_bd, t2(out_fc2_b), o3w_bd, t2(out_fc3_b))
    return out


# speedup vs baseline: 2.4482x; 2.4482x over previous
import jax
import jax.numpy as jnp
import numpy as np
from jax.experimental import pallas as pl
from jax.experimental.pallas import tpu as pltpu


def _decoder_body(T, V, Vp, DIN, H, ET,
                  w_ref, ins_ref, w1s_ref, w1r_ref, b1_ref, w2_ref, b2_ref,
                  hr_ref, hi_ref, hh_ref,
                  irw_ref, irb_ref, iiw_ref, iib_ref, inw_ref, inb_ref,
                  o1w_ref, o1b_ref, o2w_ref, o2b_ref, o3w_ref, o3b_ref,
                  out_ref):
    P = V * Vp
    H2 = 2 * H
    f32 = jnp.float32

    inv_norm = 1.0 / ((ET - 1.0) * (V - 1.0))
    dot = lambda a, b: jnp.dot(a, b, preferred_element_type=f32)

    wp = [jnp.concatenate(
              [jnp.broadcast_to(w_ref[bb, :, et:et + 1], (P, H))
               for bb in range(2)], axis=1)
          for et in range(ET)]

    ins = ins_ref[0]
    hidden = jnp.zeros((Vp, H2), dtype=f32)

    for t in range(T):
        m2w = jnp.zeros((P, H2), dtype=f32)
        for et in range(1, ET):
            s_part = dot(hidden, w1s_ref[et])
            a_part = dot(hidden, w1r_ref[et]) + b1_ref[et]
            pre = s_part[:V][:, None, :] + a_part[None, :, :]
            m = jnp.tanh(pre).reshape(P, H2)
            m2 = jnp.tanh(dot(m, w2_ref[et]) + b2_ref[et])
            m2w = m2w + m2 * wp[et]
        agg = jnp.sum(m2w.reshape(V, Vp, H2), axis=0) * inv_norm

        inp_r = dot(ins, irw_ref[...]) + irb_ref[0]
        inp_i = dot(ins, iiw_ref[...]) + iib_ref[0]
        inp_n = dot(ins, inw_ref[...]) + inb_ref[0]
        r = jax.nn.sigmoid(inp_r + dot(agg, hr_ref[...]))
        ig = jax.nn.sigmoid(inp_i + dot(agg, hi_ref[...]))
        n = jnp.tanh(inp_n + r * dot(agg, hh_ref[...]))
        hidden = (1.0 - ig) * n + ig * hidden

        p = jax.nn.relu(dot(hidden, o1w_ref[...]) + o1b_ref[0])
        p = jax.nn.relu(dot(p, o2w_ref[...]) + o2b_ref[0])
        p = dot(p, o3w_ref[...]) + o3b_ref[0]
        pred = ins + p
        out_ref[0, t] = pred[:V, :DIN]
        out_ref[1, t] = pred[:V, DIN:]
        ins = pred


def kernel(inputs, sampled_edges, msg_fc1_w, msg_fc1_b, msg_fc2_w,
           msg_fc2_b, hidden_r_w, hidden_i_w, hidden_h_w, input_r_w,
           input_r_b, input_i_w, input_i_b, input_n_w, input_n_b,
           out_fc1_w, out_fc1_b, out_fc2_w, out_fc2_b, out_fc3_w,
           out_fc3_b):
    B, T, V, DIN = inputs.shape
    H = hidden_r_w.shape[0]
    ET = msg_fc1_w.shape[0]
    Vp = (V + 7) // 8 * 8
    P = V * Vp
    NB = 2

    se4 = sampled_edges.reshape(B, V, V - 1, ET)
    se_pad = jnp.pad(se4, ((0, 0), (0, 0), (0, Vp - (V - 1)), (0, 0)))
    se_shift = jnp.roll(se_pad, 1, axis=2)
    jj = np.arange(Vp)[None, :]
    ii = np.arange(V)[:, None]
    mask_lt = jnp.asarray((jj < ii)[None, :, :, None])
    mask_gt = jnp.asarray(((jj > ii) & (jj < V))[None, :, :, None])
    w4 = (jnp.where(mask_lt, se_pad, 0.0) +
          jnp.where(mask_gt, se_shift, 0.0))
    w_dense = w4.reshape(B, P, ET)

    ins0 = jnp.pad(inputs[:, 0], ((0, 0), (0, Vp - V), (0, 0)))
    ins0 = ins0.reshape(B // NB, NB, Vp, DIN).transpose(0, 2, 1, 3)
    ins0 = ins0.reshape(B // NB, Vp, NB * DIN)

    def bd(m):
        z = jnp.zeros_like(m)
        return jnp.concatenate(
            [jnp.concatenate([m, z], axis=1),
             jnp.concatenate([z, m], axis=1)], axis=0)

    w1s_bd = jnp.stack([bd(msg_fc1_w[et, H:, :]) for et in range(ET)])
    w1r_bd = jnp.stack([bd(msg_fc1_w[et, :H, :]) for et in range(ET)])
    b1_2 = jnp.tile(msg_fc1_b, (1, NB))
    w2_bd = jnp.stack([bd(msg_fc2_w[et]) for et in range(ET)])
    b2_2 = jnp.tile(msg_fc2_b, (1, NB))
    hr_bd, hi_bd, hh_bd = bd(hidden_r_w), bd(hidden_i_w), bd(hidden_h_w)
    irw_bd, iiw_bd, inw_bd = bd(input_r_w), bd(input_i_w), bd(input_n_w)
    o1w_bd, o2w_bd, o3w_bd = bd(out_fc1_w), bd(out_fc2_w), bd(out_fc3_w)
    t2 = lambda b: jnp.tile(b.reshape(1, -1), (1, NB))

    def body(*refs):
        _decoder_body(T, V, Vp, DIN, H, ET, *refs)

    rep3 = lambda shp: pl.BlockSpec(shp, lambda b: (0, 0, 0))
    rep2 = lambda shp: pl.BlockSpec(shp, lambda b: (0, 0))
    H2 = 2 * H
    D2 = 2 * DIN

    out = pl.pallas_call(
        body,
        grid=(B // NB,),
        in_specs=[
            pl.BlockSpec((NB, P, ET), lambda b: (b, 0, 0)),
            pl.BlockSpec((1, Vp, D2), lambda b: (b, 0, 0)),
            rep3((ET, H2, H2)),
            rep3((ET, H2, H2)),
            rep2((ET, H2)),
            rep3((ET, H2, H2)),
            rep2((ET, H2)),
            rep2((H2, H2)), rep2((H2, H2)), rep2((H2, H2)),
            rep2((D2, H2)), rep2((1, H2)),
            rep2((D2, H2)), rep2((1, H2)),
            rep2((D2, H2)), rep2((1, H2)),
            rep2((H2, H2)), rep2((1, H2)),
            rep2((H2, H2)), rep2((1, H2)),
            rep2((H2, D2)), rep2((1, D2)),
        ],
        out_specs=pl.BlockSpec((NB, T, V, DIN), lambda b: (b, 0, 0, 0)),
        out_shape=jax.ShapeDtypeStruct((B, T, V, DIN), jnp.float32),
        compiler_params=pltpu.CompilerParams(
            dimension_semantics=("parallel",)),
    )(w_dense, ins0, w1s_bd, w1r_bd, b1_2, w2_bd, b2_2,
      hr_bd, hi_bd, hh_bd,
      irw_bd, t2(input_r_b), iiw_bd, t2(input_i_b), inw_bd, t2(input_n_b),
      o1w_bd, t2(out_fc1_b), o2w_bd, t2(out_fc2_b), o3w_bd, t2(out_fc3_b))
    return out


# speedup vs baseline: 2.6148x; 1.0681x over previous
import jax
import jax.numpy as jnp
import numpy as np
from jax.experimental import pallas as pl
from jax.experimental.pallas import tpu as pltpu


def _decoder_body(T, V, Vp, DIN, H, ET,
                  w_ref, ins_ref, sq_ref, inw_ref, o3w_ref, bias_ref,
                  b3_ref, out_ref):
    P = V * Vp
    H2 = 2 * H
    f32 = jnp.float32

    inv_norm = 1.0 / ((ET - 1.0) * (V - 1.0))
    dot = lambda a, b: jnp.dot(a, b, preferred_element_type=f32)

    i_hr = 3 * ET

    wp = [jnp.concatenate(
              [jnp.broadcast_to(w_ref[bb, :, et:et + 1], (P, H))
               for bb in range(2)], axis=1)
          for et in range(ET)]

    ins = ins_ref[0]
    hidden = jnp.zeros((Vp, H2), dtype=f32)

    for t in range(T):
        m2w = jnp.zeros((P, H2), dtype=f32)
        for et in range(1, ET):
            s_part = dot(hidden, sq_ref[et])
            a_part = dot(hidden, sq_ref[ET + et]) + bias_ref[et]
            pre = s_part[:V][:, None, :] + a_part[None, :, :]
            m = jnp.tanh(pre).reshape(P, H2)
            m2 = jnp.tanh(dot(m, sq_ref[2 * ET + et]) + bias_ref[ET + et])
            m2w = m2w + m2 * wp[et]
        agg = jnp.sum(m2w.reshape(V, Vp, H2), axis=0) * inv_norm

        inp_r = dot(ins, inw_ref[0]) + bias_ref[2 * ET]
        inp_i = dot(ins, inw_ref[1]) + bias_ref[2 * ET + 1]
        inp_n = dot(ins, inw_ref[2]) + bias_ref[2 * ET + 2]
        r = jax.nn.sigmoid(inp_r + dot(agg, sq_ref[i_hr]))
        ig = jax.nn.sigmoid(inp_i + dot(agg, sq_ref[i_hr + 1]))
        n = jnp.tanh(inp_n + r * dot(agg, sq_ref[i_hr + 2]))
        hidden = (1.0 - ig) * n + ig * hidden

        p = jax.nn.relu(dot(hidden, sq_ref[i_hr + 3]) + bias_ref[2 * ET + 3])
        p = jax.nn.relu(dot(p, sq_ref[i_hr + 4]) + bias_ref[2 * ET + 4])
        p = dot(p, o3w_ref[...]) + b3_ref[0]
        pred = ins + p
        out_ref[0, t] = pred[:V, :DIN]
        out_ref[1, t] = pred[:V, DIN:]
        ins = pred


def kernel(inputs, sampled_edges, msg_fc1_w, msg_fc1_b, msg_fc2_w,
           msg_fc2_b, hidden_r_w, hidden_i_w, hidden_h_w, input_r_w,
           input_r_b, input_i_w, input_i_b, input_n_w, input_n_b,
           out_fc1_w, out_fc1_b, out_fc2_w, out_fc2_b, out_fc3_w,
           out_fc3_b):
    B, T, V, DIN = inputs.shape
    H = hidden_r_w.shape[0]
    ET = msg_fc1_w.shape[0]
    Vp = (V + 7) // 8 * 8
    P = V * Vp
    NB = 2

    se4 = sampled_edges.reshape(B, V, V - 1, ET)
    se_pad = jnp.pad(se4, ((0, 0), (0, 0), (0, Vp - (V - 1)), (0, 0)))
    se_shift = jnp.roll(se_pad, 1, axis=2)
    jj = np.arange(Vp)[None, :]
    ii = np.arange(V)[:, None]
    mask_lt = jnp.asarray((jj < ii)[None, :, :, None])
    mask_gt = jnp.asarray(((jj > ii) & (jj < V))[None, :, :, None])
    w4 = (jnp.where(mask_lt, se_pad, 0.0) +
          jnp.where(mask_gt, se_shift, 0.0))
    w_dense = w4.reshape(B, P, ET)

    ins0 = jnp.pad(inputs[:, 0], ((0, 0), (0, Vp - V), (0, 0)))
    ins0 = ins0.reshape(B // NB, NB, Vp, DIN).transpose(0, 2, 1, 3)
    ins0 = ins0.reshape(B // NB, Vp, NB * DIN)

    def bd(ms):
        z = jnp.zeros_like(ms)
        top = jnp.concatenate([ms, z], axis=2)
        bot = jnp.concatenate([z, ms], axis=2)
        return jnp.concatenate([top, bot], axis=1)

    sq = jnp.concatenate(
        [msg_fc1_w[:, H:, :], msg_fc1_w[:, :H, :], msg_fc2_w,
         hidden_r_w[None], hidden_i_w[None], hidden_h_w[None],
         out_fc1_w[None], out_fc2_w[None]], axis=0)
    sq_bd = bd(sq)
    inw_bd = bd(jnp.stack([input_r_w, input_i_w, input_n_w]))
    o3w_bd = bd(out_fc3_w[None])[0]
    bias = jnp.concatenate(
        [msg_fc1_b, msg_fc2_b,
         jnp.stack([input_r_b, input_i_b, input_n_b,
                    out_fc1_b, out_fc2_b])], axis=0)
    bias2 = jnp.tile(bias, (1, NB))
    b3 = jnp.tile(out_fc3_b.reshape(1, -1), (1, NB))

    def body(*refs):
        _decoder_body(T, V, Vp, DIN, H, ET, *refs)

    NSQ = 3 * ET + 5
    H2 = 2 * H
    D2 = 2 * DIN

    out = pl.pallas_call(
        body,
        grid=(B // NB,),
        in_specs=[
            pl.BlockSpec((NB, P, ET), lambda b: (b, 0, 0)),
            pl.BlockSpec((1, Vp, D2), lambda b: (b, 0, 0)),
            pl.BlockSpec((NSQ, H2, H2), lambda b: (0, 0, 0)),
            pl.BlockSpec((3, D2, H2), lambda b: (0, 0, 0)),
            pl.BlockSpec((H2, D2), lambda b: (0, 0)),
            pl.BlockSpec((2 * ET + 5, H2), lambda b: (0, 0)),
            pl.BlockSpec((1, D2), lambda b: (0, 0)),
        ],
        out_specs=pl.BlockSpec((NB, T, V, DIN), lambda b: (b, 0, 0, 0)),
        out_shape=jax.ShapeDtypeStruct((B, T, V, DIN), jnp.float32),
        compiler_params=pltpu.CompilerParams(
            dimension_semantics=("parallel",)),
    )(w_dense, ins0, sq_bd, inw_bd, o3w_bd, bias2, b3)
    return out
